# baseline (speedup 1.0000x reference)
import dataclasses
import functools

import jax
import jax.numpy as jnp
from jax import lax
from jax.experimental import pallas as pl
from jax.experimental.pallas import tpu as pltpu
from jax.experimental.pallas import tpu_sc as plsc

B = 1024
T = 50
TP = 52
P = 20
ED = 16
TD = 16
FD = ED + TD + ED

NC = 2
NS = 16
NW = NC * NS
BW = B // NW
TEXT_PER_W = BW * TP
PREV_PER_W = BW * P
GCHUNK = 128
TCH = TEXT_PER_W // GCHUNK
PCH = PREV_PER_W // GCHUNK
PAD_ROW = 1


def _prep_body(tt_ref, tl_ref, pt_ref, plen_ref, mt_ref, mp_ref,
               rdt_ref, rdp_ref):
    tl = tl_ref[...]
    col_t = lax.broadcasted_iota(jnp.int32, (B, T), 1)
    mt_ref[:, :T] = jnp.where(col_t < tl, tt_ref[...], PAD_ROW)
    mt_ref[:, T:] = jnp.full((B, TP - T), PAD_ROW, jnp.int32)
    pl_ = plen_ref[...]
    col_p = lax.broadcasted_iota(jnp.int32, (B, P), 1)
    mp_ref[...] = jnp.where(col_p < pl_, pt_ref[...], PAD_ROW)
    rdt_ref[...] = 1.0 / jnp.maximum(tl.astype(jnp.float32), 1.0)
    rdp_ref[...] = 1.0 / jnp.maximum(pl_.astype(jnp.float32), 1.0)


def _sc_compiler_params():
    cp = pltpu.CompilerParams(use_tc_tiling_on_sc=False)
    if "needs_layout_passes" in pltpu.CompilerParams.__dataclass_fields__:
        cp = dataclasses.replace(cp, needs_layout_passes=False)
    return cp


def _sc_gather_pool(text_table, event_table, text_idx, prev_idx, e1,
                    rdt, rdp):
    mesh = plsc.VectorSubcoreMesh(core_axis_name="c", subcore_axis_name="s")

    @functools.partial(
        pl.kernel,
        mesh=mesh,
        compiler_params=_sc_compiler_params(),
        out_type=jax.ShapeDtypeStruct((B, FD), jnp.float32),
        scratch_types=[
            pltpu.VMEM((TEXT_PER_W,), jnp.int32),
            pltpu.VMEM((PREV_PER_W,), jnp.int32),
            pltpu.VMEM((BW,), jnp.int32),
            pltpu.VMEM((BW,), jnp.float32),
            pltpu.VMEM((BW,), jnp.float32),
            pltpu.VMEM((TCH, GCHUNK), jnp.int32),
            pltpu.VMEM((PCH, GCHUNK), jnp.int32),
            pltpu.VMEM((TEXT_PER_W, TD), jnp.float32),
            pltpu.VMEM((PREV_PER_W, ED), jnp.float32),
            pltpu.VMEM((BW, ED), jnp.float32),
            pltpu.VMEM((BW, TD), jnp.float32),
            pltpu.VMEM((BW, ED), jnp.float32),
            pltpu.VMEM((BW, FD), jnp.float32),
            pltpu.VMEM_SHARED((NS * BW, TD), jnp.float32),
            pltpu.VMEM_SHARED((NS * BW, ED), jnp.float32),
            pltpu.SemaphoreType.DMA,
            pltpu.SemaphoreType.DMA,
        ],
    )
    def k(tt_hbm, et_hbm, ti_hbm, pi_hbm, e1_hbm, rdt_hbm, rdp_hbm, out_hbm,
          ti_v, pi_v, e1_v, rdt_v, rdp_v, segt_v, segp_v,
          rt_v, rp_v, re_v, at_v, ap_v, mlp_v, sa_t, sa_p, sem, sem2):
        s = lax.axis_index("s")
        wid = s * NC + lax.axis_index("c")
        b0 = wid * BW
        soff = s * BW
        pltpu.sync_copy(ti_hbm.at[pl.ds(b0 * TP, TEXT_PER_W)], ti_v)
        pltpu.sync_copy(pi_hbm.at[pl.ds(b0 * P, PREV_PER_W)], pi_v)
        pltpu.sync_copy(e1_hbm.at[pl.ds(b0, BW)], e1_v)
        pltpu.sync_copy(rdt_hbm.at[pl.ds(b0, BW)], rdt_v)
        pltpu.sync_copy(rdp_hbm.at[pl.ds(b0, BW)], rdp_v)

        handles = []
        for c in range(TCH):
            handles.append(pltpu.async_copy(
                tt_hbm.at[ti_v.at[pl.ds(c * GCHUNK, GCHUNK)]],
                rt_v.at[pl.ds(c * GCHUNK, GCHUNK)], sem))
        for c in range(PCH):
            handles.append(pltpu.async_copy(
                et_hbm.at[pi_v.at[pl.ds(c * GCHUNK, GCHUNK)]],
                rp_v.at[pl.ds(c * GCHUNK, GCHUNK)], sem))
        handles.append(pltpu.async_copy(et_hbm.at[e1_v], re_v, sem))

        iota = lax.broadcasted_iota(jnp.int32, (16,), 0)
        soff_v = jnp.full((16,), soff, jnp.int32)
        tp_v = jnp.full((16,), TP, jnp.int32)
        p_v = jnp.full((16,), P, jnp.int32)
        for c in range(TCH):
            for kk in range(GCHUNK // 16):
                pos = iota + (c * GCHUNK + kk * 16)
                segt_v[c, pl.ds(kk * 16, 16)] = lax.div(pos, tp_v) + soff_v
        for c in range(PCH):
            for kk in range(GCHUNK // 16):
                pos = iota + (c * GCHUNK + kk * 16)
                segp_v[c, pl.ds(kk * 16, 16)] = lax.div(pos, p_v) + soff_v

        zero16 = jnp.zeros((16,), jnp.float32)

        @pl.loop(0, BW)
        def _(j):
            at_v[j, :] = zero16
            ap_v[j, :] = zero16

        pltpu.sync_copy(at_v, sa_t.at[pl.ds(soff, BW)])
        pltpu.sync_copy(ap_v, sa_p.at[pl.ds(soff, BW)])

        for h in handles:
            h.wait()

        handles2 = []
        for c in range(TCH):
            handles2.append(pltpu.async_copy(
                rt_v.at[pl.ds(c * GCHUNK, GCHUNK)],
                sa_t.at[segt_v.at[c]], sem2, add=True))
        for c in range(PCH):
            handles2.append(pltpu.async_copy(
                rp_v.at[pl.ds(c * GCHUNK, GCHUNK)],
                sa_p.at[segp_v.at[c]], sem2, add=True))
        for h in handles2:
            h.wait()

        pltpu.sync_copy(sa_t.at[pl.ds(soff, BW)], at_v)
        pltpu.sync_copy(sa_p.at[pl.ds(soff, BW)], ap_v)

        @pl.loop(0, BW)
        def _(j):
            mlp_v[j, pl.ds(0, ED)] = re_v[j, :]
            sc_t = plsc.load_gather(rdt_v, [jnp.full((16,), j, jnp.int32)])
            mlp_v[j, pl.ds(ED, TD)] = at_v[j, :] * sc_t
            sc_p = plsc.load_gather(rdp_v, [jnp.full((16,), j, jnp.int32)])
            mlp_v[j, pl.ds(ED + TD, ED)] = ap_v[j, :] * sc_p

        pltpu.sync_copy(mlp_v, out_hbm.at[pl.ds(b0, BW)])

    return k(text_table, event_table, text_idx, prev_idx, e1, rdt, rdp)


def _matmul_body(wt_ref, mlp_ref, b_ref, out_ref):
    acc = lax.dot_general(
        wt_ref[...].astype(jnp.bfloat16), mlp_ref[...].astype(jnp.bfloat16),
        (((0,), (1,)), ((), ())),
        preferred_element_type=jnp.float32)
    out_ref[...] = acc + b_ref[...].T


BN = 2048


def kernel(e1, e1_text_tokens, e1_text_lengths, e1prev_tokens, e1prev_lengths,
           event_table, text_table, W, b):
    EV = W.shape[0]

    mt, mp, rdt, rdp = pl.pallas_call(
        _prep_body,
        out_shape=(
            jax.ShapeDtypeStruct((B, TP), jnp.int32),
            jax.ShapeDtypeStruct((B, P), jnp.int32),
            jax.ShapeDtypeStruct((B, 1), jnp.float32),
            jax.ShapeDtypeStruct((B, 1), jnp.float32),
        ),
    )(e1_text_tokens.astype(jnp.int32),
      e1_text_lengths.reshape(B, 1).astype(jnp.int32),
      e1prev_tokens.astype(jnp.int32),
      e1prev_lengths.reshape(B, 1).astype(jnp.int32))

    mlp = _sc_gather_pool(
        text_table, event_table, mt.reshape(-1), mp.reshape(-1),
        e1.astype(jnp.int32), rdt.reshape(-1), rdp.reshape(-1))

    nblk = (EV + BN - 1) // BN
    logits_t = pl.pallas_call(
        _matmul_body,
        grid=(nblk,),
        in_specs=[
            pl.BlockSpec((FD, BN), lambda i: (0, i)),
            pl.BlockSpec((B, FD), lambda i: (0, 0)),
            pl.BlockSpec((1, BN), lambda i: (0, i)),
        ],
        out_specs=pl.BlockSpec((BN, B), lambda i: (i, 0)),
        out_shape=jax.ShapeDtypeStruct((EV, B), jnp.float32),
        compiler_params=pltpu.CompilerParams(
            dimension_semantics=("arbitrary",)),
    )(W.T, mlp, b.reshape(1, EV))
    return logits_t.T


# speedup vs baseline: 1.0008x; 1.0008x over previous
import dataclasses
import functools

import jax
import jax.numpy as jnp
from jax import lax
from jax.experimental import pallas as pl
from jax.experimental.pallas import tpu as pltpu
from jax.experimental.pallas import tpu_sc as plsc

B = 1024
T = 50
TP = 52
P = 20
ED = 16
TD = 16
FD = ED + TD + ED

NC = 2
NS = 16
NW = NC * NS
BW = B // NW
TEXT_PER_W = BW * TP
PREV_PER_W = BW * P
GCHUNK = 128
TCH = TEXT_PER_W // GCHUNK
PCH = PREV_PER_W // GCHUNK
PAD_ROW = 1


def _prep_body(tt_ref, tl_ref, pt_ref, plen_ref, mt_ref, mp_ref,
               rdt_ref, rdp_ref):
    tl = tl_ref[...]
    col_t = lax.broadcasted_iota(jnp.int32, (B, T), 1)
    mt_ref[:, :T] = jnp.where(col_t < tl, tt_ref[...], PAD_ROW)
    mt_ref[:, T:] = jnp.full((B, TP - T), PAD_ROW, jnp.int32)
    pl_ = plen_ref[...]
    col_p = lax.broadcasted_iota(jnp.int32, (B, P), 1)
    mp_ref[...] = jnp.where(col_p < pl_, pt_ref[...], PAD_ROW)
    rdt_ref[...] = 1.0 / jnp.maximum(tl.astype(jnp.float32), 1.0)
    rdp_ref[...] = 1.0 / jnp.maximum(pl_.astype(jnp.float32), 1.0)


def _sc_compiler_params():
    cp = pltpu.CompilerParams(use_tc_tiling_on_sc=False)
    if "needs_layout_passes" in pltpu.CompilerParams.__dataclass_fields__:
        cp = dataclasses.replace(cp, needs_layout_passes=False)
    return cp


def _sc_gather_pool(text_table, event_table, text_idx, prev_idx, e1,
                    rdt, rdp):
    mesh = plsc.VectorSubcoreMesh(core_axis_name="c", subcore_axis_name="s")

    @functools.partial(
        pl.kernel,
        mesh=mesh,
        compiler_params=_sc_compiler_params(),
        out_type=jax.ShapeDtypeStruct((B, FD), jnp.float32),
        scratch_types=[
            pltpu.VMEM((TEXT_PER_W,), jnp.int32),
            pltpu.VMEM((PREV_PER_W,), jnp.int32),
            pltpu.VMEM((BW,), jnp.int32),
            pltpu.VMEM((BW,), jnp.float32),
            pltpu.VMEM((BW,), jnp.float32),
            pltpu.VMEM((TEXT_PER_W, TD), jnp.float32),
            pltpu.VMEM((PREV_PER_W, ED), jnp.float32),
            pltpu.VMEM((BW, ED), jnp.float32),
            pltpu.VMEM((BW, FD), jnp.float32),
            pltpu.SemaphoreType.DMA,
        ],
    )
    def k(tt_hbm, et_hbm, ti_hbm, pi_hbm, e1_hbm, rdt_hbm, rdp_hbm, out_hbm,
          ti_v, pi_v, e1_v, rdt_v, rdp_v,
          rt_v, rp_v, re_v, mlp_v, sem):
        s = lax.axis_index("s")
        wid = s * NC + lax.axis_index("c")
        b0 = wid * BW
        pltpu.sync_copy(ti_hbm.at[pl.ds(b0 * TP, TEXT_PER_W)], ti_v)
        pltpu.sync_copy(pi_hbm.at[pl.ds(b0 * P, PREV_PER_W)], pi_v)
        pltpu.sync_copy(e1_hbm.at[pl.ds(b0, BW)], e1_v)
        pltpu.sync_copy(rdt_hbm.at[pl.ds(b0, BW)], rdt_v)
        pltpu.sync_copy(rdp_hbm.at[pl.ds(b0, BW)], rdp_v)

        handles = []
        for c in range(TCH):
            handles.append(pltpu.async_copy(
                tt_hbm.at[ti_v.at[pl.ds(c * GCHUNK, GCHUNK)]],
                rt_v.at[pl.ds(c * GCHUNK, GCHUNK)], sem))
        for c in range(PCH):
            handles.append(pltpu.async_copy(
                et_hbm.at[pi_v.at[pl.ds(c * GCHUNK, GCHUNK)]],
                rp_v.at[pl.ds(c * GCHUNK, GCHUNK)], sem))
        handles.append(pltpu.async_copy(et_hbm.at[e1_v], re_v, sem))
        for h in handles:
            h.wait()

        @pl.loop(0, BW, step=4)
        def _(j0):
            for dj in range(4):
                j = j0 + dj
                mlp_v[j, pl.ds(0, ED)] = re_v[j, :]

                tb = j * TP
                a = [jnp.zeros((TD,), jnp.float32) for _ in range(4)]
                for t in range(0, TP, 4):
                    for q in range(4):
                        a[q] = a[q] + rt_v[tb + t + q, :]
                acc = (a[0] + a[1]) + (a[2] + a[3])
                sc_t = plsc.load_gather(
                    rdt_v, [jnp.full((16,), j, jnp.int32)])
                mlp_v[j, pl.ds(ED, TD)] = acc * sc_t

                pb = j * P
                a2 = [jnp.zeros((ED,), jnp.float32) for _ in range(4)]
                for t in range(0, P, 4):
                    for q in range(4):
                        a2[q] = a2[q] + rp_v[pb + t + q, :]
                acc2 = (a2[0] + a2[1]) + (a2[2] + a2[3])
                sc_p = plsc.load_gather(
                    rdp_v, [jnp.full((16,), j, jnp.int32)])
                mlp_v[j, pl.ds(ED + TD, ED)] = acc2 * sc_p

        pltpu.sync_copy(mlp_v, out_hbm.at[pl.ds(b0, BW)])

    return k(text_table, event_table, text_idx, prev_idx, e1, rdt, rdp)


def _matmul_body(wt_ref, mlp_ref, b_ref, out_ref):
    acc = lax.dot_general(
        wt_ref[...].astype(jnp.bfloat16), mlp_ref[...].astype(jnp.bfloat16),
        (((0,), (1,)), ((), ())),
        preferred_element_type=jnp.float32)
    out_ref[...] = acc + b_ref[...].T


BN = 2048


def kernel(e1, e1_text_tokens, e1_text_lengths, e1prev_tokens, e1prev_lengths,
           event_table, text_table, W, b):
    EV = W.shape[0]

    mt, mp, rdt, rdp = pl.pallas_call(
        _prep_body,
        out_shape=(
            jax.ShapeDtypeStruct((B, TP), jnp.int32),
            jax.ShapeDtypeStruct((B, P), jnp.int32),
            jax.ShapeDtypeStruct((B, 1), jnp.float32),
            jax.ShapeDtypeStruct((B, 1), jnp.float32),
        ),
    )(e1_text_tokens.astype(jnp.int32),
      e1_text_lengths.reshape(B, 1).astype(jnp.int32),
      e1prev_tokens.astype(jnp.int32),
      e1prev_lengths.reshape(B, 1).astype(jnp.int32))

    mlp = _sc_gather_pool(
        text_table, event_table, mt.reshape(-1), mp.reshape(-1),
        e1.astype(jnp.int32), rdt.reshape(-1), rdp.reshape(-1))

    nblk = (EV + BN - 1) // BN
    logits_t = pl.pallas_call(
        _matmul_body,
        grid=(nblk,),
        in_specs=[
            pl.BlockSpec((FD, BN), lambda i: (0, i)),
            pl.BlockSpec((B, FD), lambda i: (0, 0)),
            pl.BlockSpec((1, BN), lambda i: (0, i)),
        ],
        out_specs=pl.BlockSpec((BN, B), lambda i: (i, 0)),
        out_shape=jax.ShapeDtypeStruct((EV, B), jnp.float32),
        compiler_params=pltpu.CompilerParams(
            dimension_semantics=("arbitrary",)),
    )(W.T, mlp, b.reshape(1, EV))
    return logits_t.T
